# Initial kernel scaffold; baseline (speedup 1.0000x reference)
import jax
import jax.numpy as jnp
from jax.experimental import pallas as pl


def kernel(x, W):
    raise NotImplementedError("write your pallas kernel here")


# speedup vs baseline: 1.7356x; 1.7356x over previous
import jax
import jax.numpy as jnp
from jax.experimental import pallas as pl


def _router_body(x_ref, w_ref, out_ref, logits_ref, idx_ref):
    xt = x_ref[...]
    out_ref[...] = xt
    lg = jax.lax.dot_general(
        xt, w_ref[...],
        dimension_numbers=(((1,), (1,)), ((), ())),
        preferred_element_type=jnp.float32,
        precision=jax.lax.Precision.DEFAULT,
    )
    logits_ref[...] = lg
    e = lg.shape[1]
    ids = jax.lax.broadcasted_iota(jnp.int32, lg.shape, 1)
    maxv = jnp.max(lg, axis=1, keepdims=True)
    idx_ref[...] = jnp.min(jnp.where(lg == maxv, ids, e), axis=1, keepdims=True)


def kernel(x, W):
    B, T, D = x.shape
    E = W.shape[0]
    N = B * T
    TILE = 512
    x2 = x.reshape(N, D)
    out2, logits2, idx2 = pl.pallas_call(
        _router_body,
        grid=(N // TILE,),
        in_specs=[
            pl.BlockSpec((TILE, D), lambda i: (i, 0)),
            pl.BlockSpec((E, D), lambda i: (0, 0)),
        ],
        out_specs=[
            pl.BlockSpec((TILE, D), lambda i: (i, 0)),
            pl.BlockSpec((TILE, E), lambda i: (i, 0)),
            pl.BlockSpec((TILE, 1), lambda i: (i, 0)),
        ],
        out_shape=[
            jax.ShapeDtypeStruct((N, D), jnp.float32),
            jax.ShapeDtypeStruct((N, E), jnp.float32),
            jax.ShapeDtypeStruct((N, 1), jnp.int32),
        ],
    )(x2, W)
    output = out2.reshape(B, T, D)
    logits = logits2.reshape(B, T, E)
    indices = idx2.reshape(B, T, 1)
    weights = jnp.ones((B, T, 1), jnp.float32)
    return output, logits, indices, weights
